# baseline (speedup 1.0000x reference)
import functools

import jax
import jax.numpy as jnp
from jax import lax
from jax.experimental import pallas as pl
from jax.experimental.pallas import tpu as pltpu
from jax.experimental.pallas import tpu_sc as plsc

_NUM_CORES = 2
_NUM_SUBCORES = 16
_NW = _NUM_CORES * _NUM_SUBCORES
_CHUNK = 128
_STEP = 64
_NBUF = 4
_L = 16


@functools.lru_cache(maxsize=None)
def _build(n_rows, dim):
    rows_per_w = n_rows // _NW
    chunks_per_w = rows_per_w // _CHUNK
    steps_per_chunk = _CHUNK // _STEP
    mesh = plsc.VectorSubcoreMesh(core_axis_name="c", subcore_axis_name="s")

    @functools.partial(
        pl.kernel,
        mesh=mesh,
        out_type=jax.ShapeDtypeStruct((n_rows, dim), jnp.float32),
        scratch_types=(
            [
                pltpu.VMEM((chunks_per_w, _CHUNK), jnp.int32),
                pltpu.VMEM((chunks_per_w, _CHUNK), jnp.int32),
            ]
            + [pltpu.VMEM((_STEP, 2 * dim), jnp.float32) for _ in range(_NBUF)]
            + [pltpu.VMEM((_STEP, dim), jnp.float32) for _ in range(_NBUF)]
            + [pltpu.SemaphoreType.DMA for _ in range(2 * _NBUF)]
        ),
        compiler_params=pltpu.CompilerParams(
            use_tc_tiling_on_sc=True, skip_device_barrier=True
        ),
    )
    def run(idx_hbm, table_hbm, out_hbm, idx_v, idx2_v, *bufs_and_sems):
        bufs = bufs_and_sems[:_NBUF]
        obufs = bufs_and_sems[_NBUF : 2 * _NBUF]
        gsems = bufs_and_sems[2 * _NBUF : 3 * _NBUF]
        osems = bufs_and_sems[3 * _NBUF :]
        wid = lax.axis_index("s") * _NUM_CORES + lax.axis_index("c")
        pltpu.sync_copy(idx_hbm.at[pl.ds(wid * chunks_per_w, chunks_per_w)], idx_v)
        base = wid * rows_per_w

        def halve(j, c):
            for g in range(_CHUNK // _L):
                idx2_v[j, pl.ds(g * _L, _L)] = idx_v[j, pl.ds(g * _L, _L)] >> 1
            return c

        lax.fori_loop(0, chunks_per_w, halve, 0)

        def gather(j, h, b):
            return pltpu.make_async_copy(
                table_hbm.at[idx2_v.at[j, pl.ds(h * _STEP, _STEP)]],
                bufs[b],
                gsems[b],
            )

        def store(j, h, b):
            return pltpu.make_async_copy(
                obufs[b],
                out_hbm.at[pl.ds(base + j * _CHUNK + h * _STEP, _STEP)],
                osems[b],
            )

        def repack(j, h, b):
            buf, obuf = bufs[b], obufs[b]

            def blk(i, c):
                k0 = i * _L
                off_vec = (idx_v[j, pl.ds(h * _STEP + k0, _L)] & 1) * dim
                for t in range(_L):
                    off = off_vec[t]
                    k = k0 + t
                    for c4 in range(dim // _L):
                        obuf[k, pl.ds(c4 * _L, _L)] = buf[
                            k, pl.ds(off + c4 * _L, _L)
                        ]
                return c

            lax.fori_loop(0, _STEP // _L, blk, 0)

        for b in range(_NBUF):
            gather(b // steps_per_chunk, b % steps_per_chunk, b).start()

        chunks_per_group = _NBUF // steps_per_chunk

        def loop_body(g, carry):
            j0 = g * chunks_per_group
            for b in range(_NBUF):
                j, h = j0 + b // steps_per_chunk, b % steps_per_chunk
                gather(j, h, b).wait()
                repack(j, h, b)
                store(j, h, b).start()
            for b in range(_NBUF):
                j, h = j0 + b // steps_per_chunk, b % steps_per_chunk
                store(j, h, b).wait()
                nj = j + chunks_per_group

                @pl.when(nj < chunks_per_w)
                def _():
                    gather(nj, h, b).start()

            return carry

        lax.fori_loop(0, chunks_per_w // chunks_per_group, loop_body, 0)

    return run


def kernel(token_ids, weight):
    n_rows = token_ids.size
    dim = weight.shape[1]
    idx = token_ids.reshape(n_rows // _CHUNK, _CHUNK).astype(jnp.int32)
    pair_table = weight.reshape(weight.shape[0] // 2, 2 * dim)
    out = _build(n_rows, dim)(idx, pair_table)
    return out.reshape(token_ids.shape + (dim,))
